# baseline (speedup 1.0000x reference)
import functools

import jax
import jax.numpy as jnp
from jax import lax
from jax.experimental import pallas as pl
from jax.experimental.pallas import tpu as pltpu
from jax.experimental.pallas import tpu_sc as plsc

_B = 4096
_K1 = 21
_LOG2 = 0.6931471805599453

_NC = 2
_NS = 16
_NW = _NC * _NS
_RPW = _B // _NW
_RB = 8
_CAP = 256
_TAU0 = 2.0


def _neg_log_sigmoid(d):
    return jnp.maximum(-d, 0.0) + jnp.log1p(jnp.exp(-jnp.abs(d)))


def _iter_topk_loss(x0, pos, rank_self, rows, s_ref):
    neginf = jnp.float32(-jnp.inf)
    sarange = jax.lax.broadcasted_iota(jnp.int32, (1, _K1), 1).astype(jnp.float32)
    wrow = 1.0 / jnp.log2(sarange + 2.0)

    self_in = rank_self < _K1
    w_self = jnp.sum(jnp.where(sarange == rank_self, wrow, 0.0),
                     axis=1, keepdims=True)
    loss_self = jnp.where(self_in, w_self * _LOG2, 0.0)
    mask_rows = jnp.where(self_in, _K1 - 1.0, float(_K1))

    s_ref[...] = x0

    def body(_, carry):
        r_cur, loss_row = carry
        xm = s_ref[...]
        m = jnp.max(xm, axis=1, keepdims=True)
        eq = xm == m
        c = jnp.sum(eq.astype(jnp.float32), axis=1, keepdims=True)
        s_ref[...] = jnp.where(eq, neginf, xm)
        in_win = (sarange >= r_cur) & (sarange < r_cur + c)
        wsum = jnp.sum(jnp.where(in_win, wrow, 0.0), axis=1, keepdims=True)
        f = _neg_log_sigmoid(pos - m)
        loss_row = loss_row + jnp.where(wsum > 0.0, f * wsum, 0.0)
        return r_cur + c, loss_row

    zero = jnp.zeros((rows, 1), jnp.float32)
    _, loss_row = jax.lax.fori_loop(0, _K1, body, (zero, zero))
    return loss_row - loss_self, mask_rows


_DROWS = 64


def _dense_body(x_ref, out_ref, s_ref, acc_ref):
    i = pl.program_id(0)
    x0 = x_ref[...]
    cols = jax.lax.broadcasted_iota(jnp.int32, (_DROWS, _B), 1)
    rowg = i * _DROWS + jax.lax.broadcasted_iota(jnp.int32, (_DROWS, _B), 0)
    neginf = jnp.float32(-jnp.inf)

    is_diag = cols == rowg
    pos = jnp.max(jnp.where(is_diag, x0, neginf), axis=1, keepdims=True)
    cnt_gt = jnp.sum((x0 > pos).astype(jnp.float32), axis=1, keepdims=True)
    cnt_eq = jnp.sum(((x0 == pos) & (cols < rowg)).astype(jnp.float32),
                     axis=1, keepdims=True)
    rank_self = cnt_gt + cnt_eq

    loss_row, mask_rows = _iter_topk_loss(x0, pos, rank_self, _DROWS, s_ref)

    @pl.when(i == 0)
    def _():
        acc_ref[0] = 0.0
        acc_ref[1] = 0.0

    acc_ref[0] += jnp.sum(loss_row)
    acc_ref[1] += jnp.sum(mask_rows)
    out_ref[...] = jnp.full((1, 1), acc_ref[0] / jnp.maximum(acc_ref[1], 1.0),
                            jnp.float32)


def _dense_loss(scores):
    out = pl.pallas_call(
        _dense_body,
        grid=(_B // _DROWS,),
        in_specs=[pl.BlockSpec((_DROWS, _B), lambda i: (i, 0))],
        out_specs=pl.BlockSpec((1, 1), lambda i: (0, 0)),
        out_shape=jax.ShapeDtypeStruct((1, 1), jnp.float32),
        scratch_shapes=[
            pltpu.VMEM((_DROWS, _B), jnp.float32),
            pltpu.SMEM((2,), jnp.float32),
        ],
    )(scores)
    return jnp.reshape(out, ())


_DB = 256


def _diag_body(x_ref, out_ref):
    x = x_ref[...]
    r = jax.lax.broadcasted_iota(jnp.int32, (_DB, _DB), 0)
    c = jax.lax.broadcasted_iota(jnp.int32, (_DB, _DB), 1)
    out_ref[...] = jnp.max(jnp.where(r == c, x, -jnp.inf), axis=1,
                           keepdims=True)


def _diag(scores):
    return pl.pallas_call(
        _diag_body,
        grid=(_B // _DB,),
        in_specs=[pl.BlockSpec((_DB, _DB), lambda i: (i, i))],
        out_specs=pl.BlockSpec((_DB, 1), lambda i: (i, 0)),
        out_shape=jax.ShapeDtypeStruct((_B, 1), jnp.float32),
    )(scores)


def _sc_compact_body(scores_hbm, vals_hbm, cols_hbm, cnt_hbm,
                     rowbuf, vbuf, cbuf, nbuf):
    wid = lax.axis_index("s") * _NC + lax.axis_index("c")
    base = wid * _RPW
    lane = lax.iota(jnp.int32, 16)
    tau = jnp.float32(_TAU0)

    def batch_body(bi, _):
        r0 = base + bi * _RB
        pltpu.sync_copy(scores_hbm.at[pl.ds(r0, _RB)], rowbuf)

        for r in range(_RB):
            rsplat = jnp.full((16,), r, jnp.int32)

            def vbody(j, off_vec):
                v = rowbuf[r, pl.ds(j * 16, 16)]
                colv = lane + j * 16
                mask = v > tau
                mi = mask.astype(jnp.int32)
                pos_idx = off_vec + jnp.cumsum(mi) - 1
                m2 = jnp.logical_and(mask, pos_idx < _CAP)
                pos_c = jnp.minimum(pos_idx, _CAP - 1)
                plsc.store_scatter(vbuf, (rsplat, pos_c), v, mask=m2)
                plsc.store_scatter(cbuf, (rsplat, pos_c), colv, mask=m2)
                return off_vec + plsc.all_reduce_population_count(mask)

            off_vec = lax.fori_loop(0, _B // 16, vbody,
                                    jnp.zeros((16,), jnp.int32))
            plsc.store_scatter(nbuf,
                               (jnp.zeros((16,), jnp.int32),
                                jnp.full((16,), bi * _RB + r, jnp.int32)),
                               off_vec, mask=lane == 0)

        pltpu.sync_copy(vbuf, vals_hbm.at[pl.ds(r0, _RB)])
        pltpu.sync_copy(cbuf, cols_hbm.at[pl.ds(r0, _RB)])
        return 0

    lax.fori_loop(0, _RPW // _RB, batch_body, 0)
    pltpu.sync_copy(nbuf, cnt_hbm.at[pl.ds(wid, 1)])


@functools.cache
def _make_sc_compact():
    return pl.kernel(
        _sc_compact_body,
        out_type=[
            jax.ShapeDtypeStruct((_B, _CAP), jnp.float32),
            jax.ShapeDtypeStruct((_B, _CAP), jnp.int32),
            jax.ShapeDtypeStruct((_NW, _RPW), jnp.int32),
        ],
        mesh=plsc.VectorSubcoreMesh(core_axis_name="c", subcore_axis_name="s"),
        compiler_params=pltpu.CompilerParams(needs_layout_passes=False),
        scratch_types=[
            pltpu.VMEM((_RB, _B), jnp.float32),
            pltpu.VMEM((_RB, _CAP), jnp.float32),
            pltpu.VMEM((_RB, _CAP), jnp.int32),
            pltpu.VMEM((1, _RPW), jnp.int32),
        ],
    )


_PR = 256


def _cand_body(vals_ref, cols_ref, cnt_ref, pos_ref, out_ref, bad_ref,
               s_ref, acc_ref):
    i = pl.program_id(0)
    vals = vals_ref[...]
    cols = cols_ref[...]
    cnt = cnt_ref[...].astype(jnp.int32)
    pos = pos_ref[...]
    neginf = jnp.float32(-jnp.inf)

    cpos = jax.lax.broadcasted_iota(jnp.int32, (_PR, _CAP), 1)
    valid = cpos < cnt
    v = jnp.where(valid, vals, neginf)
    rowg = i * _PR + jax.lax.broadcasted_iota(jnp.int32, (_PR, _CAP), 0)

    gt = jnp.sum((v > pos).astype(jnp.float32), axis=1, keepdims=True)
    eqb = jnp.sum(((v == pos) & (cols < rowg)).astype(jnp.float32),
                  axis=1, keepdims=True)
    rank_self = jnp.where(pos > _TAU0, gt + eqb, jnp.float32(_B))

    loss_row, mask_rows = _iter_topk_loss(v, pos, rank_self, _PR, s_ref)
    bad_row = (cnt < _K1) | (cnt > _CAP)

    @pl.when(i == 0)
    def _():
        acc_ref[0] = 0.0
        acc_ref[1] = 0.0
        acc_ref[2] = 0.0

    acc_ref[0] += jnp.sum(loss_row)
    acc_ref[1] += jnp.sum(mask_rows)
    acc_ref[2] += jnp.sum(bad_row.astype(jnp.float32))
    out_ref[...] = jnp.full((1, 1), acc_ref[0] / jnp.maximum(acc_ref[1], 1.0),
                            jnp.float32)
    bad_ref[...] = jnp.full((1, 1), acc_ref[2], jnp.float32)


def _cand_loss(vals, cols, cnt, pos):
    return pl.pallas_call(
        _cand_body,
        grid=(_B // _PR,),
        in_specs=[
            pl.BlockSpec((_PR, _CAP), lambda i: (i, 0)),
            pl.BlockSpec((_PR, _CAP), lambda i: (i, 0)),
            pl.BlockSpec((_PR, 1), lambda i: (i, 0)),
            pl.BlockSpec((_PR, 1), lambda i: (i, 0)),
        ],
        out_specs=[
            pl.BlockSpec((1, 1), lambda i: (0, 0)),
            pl.BlockSpec((1, 1), lambda i: (0, 0)),
        ],
        out_shape=[
            jax.ShapeDtypeStruct((1, 1), jnp.float32),
            jax.ShapeDtypeStruct((1, 1), jnp.float32),
        ],
        scratch_shapes=[
            pltpu.VMEM((_PR, _CAP), jnp.float32),
            pltpu.SMEM((3,), jnp.float32),
        ],
    )(vals, cols, cnt, pos)


@jax.jit
def kernel(scores):
    pos = _diag(scores)
    vals, cols, cnt2d = _make_sc_compact()(scores)
    cnt = cnt2d.reshape(_B, 1)
    fast, badf = _cand_loss(vals, cols, cnt, pos)
    fast_s = jnp.reshape(fast, ())
    bad = jnp.reshape(badf, ()) > 0.0
    return lax.cond(bad, _dense_loss, lambda s: fast_s, scores)


# speedup vs baseline: 1.6356x; 1.6356x over previous
import functools

import jax
import jax.numpy as jnp
from jax import lax
from jax.experimental import pallas as pl
from jax.experimental.pallas import tpu as pltpu
from jax.experimental.pallas import tpu_sc as plsc

_B = 4096
_K1 = 21
_LOG2 = 0.6931471805599453

_NC = 2
_NS = 16
_NW = _NC * _NS
_RPW = _B // _NW
_RB = 8
_CAP = 256
_TAU0 = 2.0


def _neg_log_sigmoid(d):
    return jnp.maximum(-d, 0.0) + jnp.log1p(jnp.exp(-jnp.abs(d)))


def _iter_topk_loss(x0, pos, rank_self, rows, s_ref):
    neginf = jnp.float32(-jnp.inf)
    sarange = jax.lax.broadcasted_iota(jnp.int32, (1, _K1), 1).astype(jnp.float32)
    wrow = 1.0 / jnp.log2(sarange + 2.0)

    self_in = rank_self < _K1
    w_self = jnp.sum(jnp.where(sarange == rank_self, wrow, 0.0),
                     axis=1, keepdims=True)
    loss_self = jnp.where(self_in, w_self * _LOG2, 0.0)
    mask_rows = jnp.where(self_in, _K1 - 1.0, float(_K1))

    s_ref[...] = x0

    def body(_, carry):
        r_cur, loss_row = carry
        xm = s_ref[...]
        m = jnp.max(xm, axis=1, keepdims=True)
        eq = xm == m
        c = jnp.sum(eq.astype(jnp.float32), axis=1, keepdims=True)
        s_ref[...] = jnp.where(eq, neginf, xm)
        in_win = (sarange >= r_cur) & (sarange < r_cur + c)
        wsum = jnp.sum(jnp.where(in_win, wrow, 0.0), axis=1, keepdims=True)
        f = _neg_log_sigmoid(pos - m)
        loss_row = loss_row + jnp.where(wsum > 0.0, f * wsum, 0.0)
        return r_cur + c, loss_row

    zero = jnp.zeros((rows, 1), jnp.float32)
    _, loss_row = jax.lax.fori_loop(0, _K1, body, (zero, zero))
    return loss_row - loss_self, mask_rows


_DROWS = 64


def _dense_body(x_ref, out_ref, s_ref, acc_ref):
    i = pl.program_id(0)
    x0 = x_ref[...]
    cols = jax.lax.broadcasted_iota(jnp.int32, (_DROWS, _B), 1)
    rowg = i * _DROWS + jax.lax.broadcasted_iota(jnp.int32, (_DROWS, _B), 0)
    neginf = jnp.float32(-jnp.inf)

    is_diag = cols == rowg
    pos = jnp.max(jnp.where(is_diag, x0, neginf), axis=1, keepdims=True)
    cnt_gt = jnp.sum((x0 > pos).astype(jnp.float32), axis=1, keepdims=True)
    cnt_eq = jnp.sum(((x0 == pos) & (cols < rowg)).astype(jnp.float32),
                     axis=1, keepdims=True)
    rank_self = cnt_gt + cnt_eq

    loss_row, mask_rows = _iter_topk_loss(x0, pos, rank_self, _DROWS, s_ref)

    @pl.when(i == 0)
    def _():
        acc_ref[0] = 0.0
        acc_ref[1] = 0.0

    acc_ref[0] += jnp.sum(loss_row)
    acc_ref[1] += jnp.sum(mask_rows)
    out_ref[...] = jnp.full((1, 1), acc_ref[0] / jnp.maximum(acc_ref[1], 1.0),
                            jnp.float32)


def _dense_loss(scores):
    out = pl.pallas_call(
        _dense_body,
        grid=(_B // _DROWS,),
        in_specs=[pl.BlockSpec((_DROWS, _B), lambda i: (i, 0))],
        out_specs=pl.BlockSpec((1, 1), lambda i: (0, 0)),
        out_shape=jax.ShapeDtypeStruct((1, 1), jnp.float32),
        scratch_shapes=[
            pltpu.VMEM((_DROWS, _B), jnp.float32),
            pltpu.SMEM((2,), jnp.float32),
        ],
    )(scores)
    return jnp.reshape(out, ())


_DB = 256


def _diag_body(x_ref, out_ref):
    x = x_ref[...]
    r = jax.lax.broadcasted_iota(jnp.int32, (_DB, _DB), 0)
    c = jax.lax.broadcasted_iota(jnp.int32, (_DB, _DB), 1)
    out_ref[...] = jnp.max(jnp.where(r == c, x, -jnp.inf), axis=1,
                           keepdims=True)


def _diag(scores):
    return pl.pallas_call(
        _diag_body,
        grid=(_B // _DB,),
        in_specs=[pl.BlockSpec((_DB, _DB), lambda i: (i, i))],
        out_specs=pl.BlockSpec((_DB, 1), lambda i: (i, 0)),
        out_shape=jax.ShapeDtypeStruct((_B, 1), jnp.float32),
    )(scores)


def _sc_compact_body(scores_hbm, vals_hbm, cols_hbm, cnt_hbm,
                     rowbuf, vbuf, cbuf, nbuf):
    wid = lax.axis_index("s") * _NC + lax.axis_index("c")
    base = wid * _RPW
    lane = lax.iota(jnp.int32, 16)
    tau = jnp.float32(_TAU0)

    def batch_body(bi, _):
        r0 = base + bi * _RB
        pltpu.sync_copy(scores_hbm.at[pl.ds(r0, _RB)], rowbuf)

        for r in range(_RB):
            def vbody(j4, off):
                vs, colvs, masks, cs = [], [], [], []
                for u in range(4):
                    j = j4 * 4 + u
                    v = rowbuf[r, pl.ds(j * 16, 16)]
                    mask = v > tau
                    vs.append(v)
                    colvs.append(lane + j * 16)
                    masks.append(mask)
                    cs.append(plsc.all_reduce_population_count(mask)[0])
                for u in range(4):
                    offc = jnp.minimum(off, _CAP - 16)
                    m2 = jnp.logical_and(masks[u], off < _CAP - 15)
                    plsc.store_compressed(vbuf.at[r, pl.ds(offc, 16)],
                                          vs[u], mask=m2)
                    plsc.store_compressed(cbuf.at[r, pl.ds(offc, 16)],
                                          colvs[u], mask=m2)
                    off = off + cs[u]
                return off

            off = lax.fori_loop(0, _B // 64, vbody, jnp.int32(0))
            plsc.store_scatter(nbuf,
                               (jnp.zeros((16,), jnp.int32),
                                jnp.full((16,), bi * _RB + r, jnp.int32)),
                               jnp.full((16,), off, jnp.int32),
                               mask=lane == 0)

        pltpu.sync_copy(vbuf, vals_hbm.at[pl.ds(r0, _RB)])
        pltpu.sync_copy(cbuf, cols_hbm.at[pl.ds(r0, _RB)])
        return 0

    lax.fori_loop(0, _RPW // _RB, batch_body, 0)
    pltpu.sync_copy(nbuf, cnt_hbm.at[pl.ds(wid, 1)])


@functools.cache
def _make_sc_compact():
    return pl.kernel(
        _sc_compact_body,
        out_type=[
            jax.ShapeDtypeStruct((_B, _CAP), jnp.float32),
            jax.ShapeDtypeStruct((_B, _CAP), jnp.int32),
            jax.ShapeDtypeStruct((_NW, _RPW), jnp.int32),
        ],
        mesh=plsc.VectorSubcoreMesh(core_axis_name="c", subcore_axis_name="s"),
        compiler_params=pltpu.CompilerParams(needs_layout_passes=False),
        scratch_types=[
            pltpu.VMEM((_RB, _B), jnp.float32),
            pltpu.VMEM((_RB, _CAP), jnp.float32),
            pltpu.VMEM((_RB, _CAP), jnp.int32),
            pltpu.VMEM((1, _RPW), jnp.int32),
        ],
    )


_PR = 256


def _cand_body(vals_ref, cols_ref, cnt_ref, pos_ref, out_ref, bad_ref,
               s_ref, acc_ref):
    i = pl.program_id(0)
    vals = vals_ref[...]
    cols = cols_ref[...]
    cnt = cnt_ref[...].astype(jnp.int32)
    pos = pos_ref[...]
    neginf = jnp.float32(-jnp.inf)

    cpos = jax.lax.broadcasted_iota(jnp.int32, (_PR, _CAP), 1)
    valid = cpos < cnt
    v = jnp.where(valid, vals, neginf)
    rowg = i * _PR + jax.lax.broadcasted_iota(jnp.int32, (_PR, _CAP), 0)

    gt = jnp.sum((v > pos).astype(jnp.float32), axis=1, keepdims=True)
    eqb = jnp.sum(((v == pos) & (cols < rowg)).astype(jnp.float32),
                  axis=1, keepdims=True)
    rank_self = jnp.where(pos > _TAU0, gt + eqb, jnp.float32(_B))

    loss_row, mask_rows = _iter_topk_loss(v, pos, rank_self, _PR, s_ref)
    bad_row = (cnt < _K1) | (cnt > _CAP - 16)

    @pl.when(i == 0)
    def _():
        acc_ref[0] = 0.0
        acc_ref[1] = 0.0
        acc_ref[2] = 0.0

    acc_ref[0] += jnp.sum(loss_row)
    acc_ref[1] += jnp.sum(mask_rows)
    acc_ref[2] += jnp.sum(bad_row.astype(jnp.float32))
    out_ref[...] = jnp.full((1, 1), acc_ref[0] / jnp.maximum(acc_ref[1], 1.0),
                            jnp.float32)
    bad_ref[...] = jnp.full((1, 1), acc_ref[2], jnp.float32)


def _cand_loss(vals, cols, cnt, pos):
    return pl.pallas_call(
        _cand_body,
        grid=(_B // _PR,),
        in_specs=[
            pl.BlockSpec((_PR, _CAP), lambda i: (i, 0)),
            pl.BlockSpec((_PR, _CAP), lambda i: (i, 0)),
            pl.BlockSpec((_PR, 1), lambda i: (i, 0)),
            pl.BlockSpec((_PR, 1), lambda i: (i, 0)),
        ],
        out_specs=[
            pl.BlockSpec((1, 1), lambda i: (0, 0)),
            pl.BlockSpec((1, 1), lambda i: (0, 0)),
        ],
        out_shape=[
            jax.ShapeDtypeStruct((1, 1), jnp.float32),
            jax.ShapeDtypeStruct((1, 1), jnp.float32),
        ],
        scratch_shapes=[
            pltpu.VMEM((_PR, _CAP), jnp.float32),
            pltpu.SMEM((3,), jnp.float32),
        ],
    )(vals, cols, cnt, pos)


@jax.jit
def kernel(scores):
    pos = _diag(scores)
    vals, cols, cnt2d = _make_sc_compact()(scores)
    cnt = cnt2d.reshape(_B, 1)
    fast, badf = _cand_loss(vals, cols, cnt, pos)
    fast_s = jnp.reshape(fast, ())
    bad = jnp.reshape(badf, ()) > 0.0
    return lax.cond(bad, _dense_loss, lambda s: fast_s, scores)


# speedup vs baseline: 1.9888x; 1.2160x over previous
import functools

import jax
import jax.numpy as jnp
from jax import lax
from jax.experimental import pallas as pl
from jax.experimental.pallas import tpu as pltpu
from jax.experimental.pallas import tpu_sc as plsc

_B = 4096
_K1 = 21
_LOG2 = 0.6931471805599453

_NC = 2
_NS = 16
_NW = _NC * _NS
_RPW = _B // _NW
_RB = 8
_CAP = 256
_TAU0 = 2.0


def _neg_log_sigmoid(d):
    return jnp.maximum(-d, 0.0) + jnp.log1p(jnp.exp(-jnp.abs(d)))


def _iter_topk_loss(x0, pos, rank_self, rows, s_ref):
    neginf = jnp.float32(-jnp.inf)
    sarange = jax.lax.broadcasted_iota(jnp.int32, (1, _K1), 1).astype(jnp.float32)
    wrow = 1.0 / jnp.log2(sarange + 2.0)

    self_in = rank_self < _K1
    w_self = jnp.sum(jnp.where(sarange == rank_self, wrow, 0.0),
                     axis=1, keepdims=True)
    loss_self = jnp.where(self_in, w_self * _LOG2, 0.0)
    mask_rows = jnp.where(self_in, _K1 - 1.0, float(_K1))

    s_ref[...] = x0

    def body(_, carry):
        r_cur, loss_row = carry
        xm = s_ref[...]
        m = jnp.max(xm, axis=1, keepdims=True)
        eq = xm == m
        c = jnp.sum(eq.astype(jnp.float32), axis=1, keepdims=True)
        s_ref[...] = jnp.where(eq, neginf, xm)
        in_win = (sarange >= r_cur) & (sarange < r_cur + c)
        wsum = jnp.sum(jnp.where(in_win, wrow, 0.0), axis=1, keepdims=True)
        f = _neg_log_sigmoid(pos - m)
        loss_row = loss_row + jnp.where(wsum > 0.0, f * wsum, 0.0)
        return r_cur + c, loss_row

    zero = jnp.zeros((rows, 1), jnp.float32)
    _, loss_row = jax.lax.fori_loop(0, _K1, body, (zero, zero))
    return loss_row - loss_self, mask_rows


_DROWS = 64


def _dense_body(x_ref, out_ref, s_ref, acc_ref):
    i = pl.program_id(0)
    x0 = x_ref[...]
    cols = jax.lax.broadcasted_iota(jnp.int32, (_DROWS, _B), 1)
    rowg = i * _DROWS + jax.lax.broadcasted_iota(jnp.int32, (_DROWS, _B), 0)
    neginf = jnp.float32(-jnp.inf)

    is_diag = cols == rowg
    pos = jnp.max(jnp.where(is_diag, x0, neginf), axis=1, keepdims=True)
    cnt_gt = jnp.sum((x0 > pos).astype(jnp.float32), axis=1, keepdims=True)
    cnt_eq = jnp.sum(((x0 == pos) & (cols < rowg)).astype(jnp.float32),
                     axis=1, keepdims=True)
    rank_self = cnt_gt + cnt_eq

    loss_row, mask_rows = _iter_topk_loss(x0, pos, rank_self, _DROWS, s_ref)

    @pl.when(i == 0)
    def _():
        acc_ref[0] = 0.0
        acc_ref[1] = 0.0

    acc_ref[0] += jnp.sum(loss_row)
    acc_ref[1] += jnp.sum(mask_rows)
    out_ref[...] = jnp.full((1, 1), acc_ref[0] / jnp.maximum(acc_ref[1], 1.0),
                            jnp.float32)


def _dense_loss(scores):
    out = pl.pallas_call(
        _dense_body,
        grid=(_B // _DROWS,),
        in_specs=[pl.BlockSpec((_DROWS, _B), lambda i: (i, 0))],
        out_specs=pl.BlockSpec((1, 1), lambda i: (0, 0)),
        out_shape=jax.ShapeDtypeStruct((1, 1), jnp.float32),
        scratch_shapes=[
            pltpu.VMEM((_DROWS, _B), jnp.float32),
            pltpu.SMEM((2,), jnp.float32),
        ],
    )(scores)
    return jnp.reshape(out, ())


_DB = 256


def _diag_body(x_ref, out_ref):
    x = x_ref[...]
    r = jax.lax.broadcasted_iota(jnp.int32, (_DB, _DB), 0)
    c = jax.lax.broadcasted_iota(jnp.int32, (_DB, _DB), 1)
    out_ref[...] = jnp.max(jnp.where(r == c, x, -jnp.inf), axis=1,
                           keepdims=True)


def _diag(scores):
    return pl.pallas_call(
        _diag_body,
        grid=(_B // _DB,),
        in_specs=[pl.BlockSpec((_DB, _DB), lambda i: (i, i))],
        out_specs=pl.BlockSpec((_DB, 1), lambda i: (i, 0)),
        out_shape=jax.ShapeDtypeStruct((_B, 1), jnp.float32),
    )(scores)


def _sc_compact_body(scores_hbm, vals_hbm, cols_hbm, cnt_hbm,
                     rowbuf, vbuf, cbuf, nbuf):
    wid = lax.axis_index("s") * _NC + lax.axis_index("c")
    base = wid * _RPW
    lane = lax.iota(jnp.int32, 16)
    tau = jnp.float32(_TAU0)

    def batch_body(bi, _):
        r0 = base + bi * _RB
        pltpu.sync_copy(scores_hbm.at[pl.ds(r0, _RB)], rowbuf)

        for r in range(_RB):
            def vbody(j4, off):
                vs, colvs, masks, cs = [], [], [], []
                for u in range(8):
                    j = j4 * 8 + u
                    v = rowbuf[r, pl.ds(j * 16, 16)]
                    mask = v > tau
                    vs.append(v)
                    colvs.append(lane + j * 16)
                    masks.append(mask)
                    cs.append(plsc.all_reduce_population_count(mask)[0])
                for u in range(8):
                    offc = jnp.minimum(off, _CAP - 16)
                    m2 = jnp.logical_and(masks[u], off < _CAP - 15)
                    plsc.store_compressed(vbuf.at[r, pl.ds(offc, 16)],
                                          vs[u], mask=m2)
                    plsc.store_compressed(cbuf.at[r, pl.ds(offc, 16)],
                                          colvs[u], mask=m2)
                    off = off + cs[u]
                return off

            off = lax.fori_loop(0, _B // 128, vbody, jnp.int32(0))
            plsc.store_scatter(nbuf,
                               (jnp.zeros((16,), jnp.int32),
                                jnp.full((16,), bi * _RB + r, jnp.int32)),
                               jnp.full((16,), off, jnp.int32),
                               mask=lane == 0)

        pltpu.sync_copy(vbuf, vals_hbm.at[pl.ds(r0, _RB)])
        pltpu.sync_copy(cbuf, cols_hbm.at[pl.ds(r0, _RB)])
        return 0

    lax.fori_loop(0, _RPW // _RB, batch_body, 0)
    pltpu.sync_copy(nbuf, cnt_hbm.at[pl.ds(wid, 1)])


@functools.cache
def _make_sc_compact():
    return pl.kernel(
        _sc_compact_body,
        out_type=[
            jax.ShapeDtypeStruct((_B, _CAP), jnp.float32),
            jax.ShapeDtypeStruct((_B, _CAP), jnp.int32),
            jax.ShapeDtypeStruct((_NW, _RPW), jnp.int32),
        ],
        mesh=plsc.VectorSubcoreMesh(core_axis_name="c", subcore_axis_name="s"),
        compiler_params=pltpu.CompilerParams(needs_layout_passes=False),
        scratch_types=[
            pltpu.VMEM((_RB, _B), jnp.float32),
            pltpu.VMEM((_RB, _CAP), jnp.float32),
            pltpu.VMEM((_RB, _CAP), jnp.int32),
            pltpu.VMEM((1, _RPW), jnp.int32),
        ],
    )


_PR = 256


def _cand_body(vals_ref, cols_ref, cnt_ref, pos_ref, out_ref, bad_ref,
               s_ref, acc_ref):
    i = pl.program_id(0)
    vals = vals_ref[...]
    cols = cols_ref[...]
    cnt = cnt_ref[...].astype(jnp.int32)
    pos = pos_ref[...]
    neginf = jnp.float32(-jnp.inf)

    cpos = jax.lax.broadcasted_iota(jnp.int32, (_PR, _CAP), 1)
    valid = cpos < cnt
    v = jnp.where(valid, vals, neginf)
    rowg = i * _PR + jax.lax.broadcasted_iota(jnp.int32, (_PR, _CAP), 0)

    gt = jnp.sum((v > pos).astype(jnp.float32), axis=1, keepdims=True)
    eqb = jnp.sum(((v == pos) & (cols < rowg)).astype(jnp.float32),
                  axis=1, keepdims=True)
    rank_self = jnp.where(pos > _TAU0, gt + eqb, jnp.float32(_B))

    loss_row, mask_rows = _iter_topk_loss(v, pos, rank_self, _PR, s_ref)
    bad_row = (cnt < _K1) | (cnt > _CAP - 16)

    @pl.when(i == 0)
    def _():
        acc_ref[0] = 0.0
        acc_ref[1] = 0.0
        acc_ref[2] = 0.0

    acc_ref[0] += jnp.sum(loss_row)
    acc_ref[1] += jnp.sum(mask_rows)
    acc_ref[2] += jnp.sum(bad_row.astype(jnp.float32))
    out_ref[...] = jnp.full((1, 1), acc_ref[0] / jnp.maximum(acc_ref[1], 1.0),
                            jnp.float32)
    bad_ref[...] = jnp.full((1, 1), acc_ref[2], jnp.float32)


def _cand_loss(vals, cols, cnt, pos):
    return pl.pallas_call(
        _cand_body,
        grid=(_B // _PR,),
        in_specs=[
            pl.BlockSpec((_PR, _CAP), lambda i: (i, 0)),
            pl.BlockSpec((_PR, _CAP), lambda i: (i, 0)),
            pl.BlockSpec((_PR, 1), lambda i: (i, 0)),
            pl.BlockSpec((_PR, 1), lambda i: (i, 0)),
        ],
        out_specs=[
            pl.BlockSpec((1, 1), lambda i: (0, 0)),
            pl.BlockSpec((1, 1), lambda i: (0, 0)),
        ],
        out_shape=[
            jax.ShapeDtypeStruct((1, 1), jnp.float32),
            jax.ShapeDtypeStruct((1, 1), jnp.float32),
        ],
        scratch_shapes=[
            pltpu.VMEM((_PR, _CAP), jnp.float32),
            pltpu.SMEM((3,), jnp.float32),
        ],
    )(vals, cols, cnt, pos)


@jax.jit
def kernel(scores):
    pos = _diag(scores)
    vals, cols, cnt2d = _make_sc_compact()(scores)
    cnt = cnt2d.reshape(_B, 1)
    fast, badf = _cand_loss(vals, cols, cnt, pos)
    fast_s = jnp.reshape(fast, ())
    bad = jnp.reshape(badf, ()) > 0.0
    return lax.cond(bad, _dense_loss, lambda s: fast_s, scores)


# speedup vs baseline: 2.2420x; 1.1273x over previous
import functools

import jax
import jax.numpy as jnp
from jax import lax
from jax.experimental import pallas as pl
from jax.experimental.pallas import tpu as pltpu
from jax.experimental.pallas import tpu_sc as plsc

_B = 4096
_K1 = 21
_LOG2 = 0.6931471805599453

_NC = 2
_NS = 16
_NW = _NC * _NS
_RPW = _B // _NW
_RB = 8
_CAP = 256
_TAU0 = 2.0


def _neg_log_sigmoid(d):
    return jnp.maximum(-d, 0.0) + jnp.log1p(jnp.exp(-jnp.abs(d)))


def _iter_topk_loss(x0, pos, rank_self, rows, s_ref):
    neginf = jnp.float32(-jnp.inf)
    sarange = jax.lax.broadcasted_iota(jnp.int32, (1, _K1), 1).astype(jnp.float32)
    wrow = 1.0 / jnp.log2(sarange + 2.0)

    self_in = rank_self < _K1
    w_self = jnp.sum(jnp.where(sarange == rank_self, wrow, 0.0),
                     axis=1, keepdims=True)
    loss_self = jnp.where(self_in, w_self * _LOG2, 0.0)
    mask_rows = jnp.where(self_in, _K1 - 1.0, float(_K1))

    s_ref[...] = x0

    def body(_, carry):
        r_cur, loss_row = carry
        xm = s_ref[...]
        m = jnp.max(xm, axis=1, keepdims=True)
        eq = xm == m
        c = jnp.sum(eq.astype(jnp.float32), axis=1, keepdims=True)
        s_ref[...] = jnp.where(eq, neginf, xm)
        in_win = (sarange >= r_cur) & (sarange < r_cur + c)
        wsum = jnp.sum(jnp.where(in_win, wrow, 0.0), axis=1, keepdims=True)
        f = _neg_log_sigmoid(pos - m)
        loss_row = loss_row + jnp.where(wsum > 0.0, f * wsum, 0.0)
        return r_cur + c, loss_row

    zero = jnp.zeros((rows, 1), jnp.float32)
    _, loss_row = jax.lax.fori_loop(0, _K1, body, (zero, zero))
    return loss_row - loss_self, mask_rows


_DROWS = 64


def _dense_body(x_ref, out_ref, s_ref, acc_ref):
    i = pl.program_id(0)
    x0 = x_ref[...]
    cols = jax.lax.broadcasted_iota(jnp.int32, (_DROWS, _B), 1)
    rowg = i * _DROWS + jax.lax.broadcasted_iota(jnp.int32, (_DROWS, _B), 0)
    neginf = jnp.float32(-jnp.inf)

    is_diag = cols == rowg
    pos = jnp.max(jnp.where(is_diag, x0, neginf), axis=1, keepdims=True)
    cnt_gt = jnp.sum((x0 > pos).astype(jnp.float32), axis=1, keepdims=True)
    cnt_eq = jnp.sum(((x0 == pos) & (cols < rowg)).astype(jnp.float32),
                     axis=1, keepdims=True)
    rank_self = cnt_gt + cnt_eq

    loss_row, mask_rows = _iter_topk_loss(x0, pos, rank_self, _DROWS, s_ref)

    @pl.when(i == 0)
    def _():
        acc_ref[0] = 0.0
        acc_ref[1] = 0.0

    acc_ref[0] += jnp.sum(loss_row)
    acc_ref[1] += jnp.sum(mask_rows)
    out_ref[...] = jnp.full((1, 1), acc_ref[0] / jnp.maximum(acc_ref[1], 1.0),
                            jnp.float32)


def _dense_loss(scores):
    out = pl.pallas_call(
        _dense_body,
        grid=(_B // _DROWS,),
        in_specs=[pl.BlockSpec((_DROWS, _B), lambda i: (i, 0))],
        out_specs=pl.BlockSpec((1, 1), lambda i: (0, 0)),
        out_shape=jax.ShapeDtypeStruct((1, 1), jnp.float32),
        scratch_shapes=[
            pltpu.VMEM((_DROWS, _B), jnp.float32),
            pltpu.SMEM((2,), jnp.float32),
        ],
    )(scores)
    return jnp.reshape(out, ())


def _sc_compact_body(scores_hbm, vals_hbm, rowA, rowB, vbuf, semA, semB):
    wid = lax.axis_index("s") * _NC + lax.axis_index("c")
    base = wid * _RPW
    tau = jnp.float32(_TAU0)
    nb = _RPW // _RB

    def _start(bi, buf, sem):
        pltpu.async_copy(scores_hbm.at[pl.ds(base + bi * _RB, _RB)], buf, sem)

    def _wait(bi, buf, sem):
        pltpu.make_async_copy(scores_hbm.at[pl.ds(base + bi * _RB, _RB)],
                              buf, sem).wait()

    def _do_batch(bi, rowbuf):
        for r in range(_RB):
            def vbody(j8, off):
                vs, masks, cs = [], [], []
                for u in range(8):
                    j = j8 * 8 + u
                    v = rowbuf[r, pl.ds(j * 16, 16)]
                    mask = v > tau
                    vs.append(v)
                    masks.append(mask)
                    cs.append(plsc.all_reduce_population_count(mask)[0])
                for u in range(8):
                    offc = jnp.minimum(off, _CAP - 16)
                    m2 = jnp.logical_and(masks[u], off < _CAP - 15)
                    plsc.store_compressed(vbuf.at[r, pl.ds(offc, 16)],
                                          vs[u], mask=m2)
                    off = off + cs[u]
                return off

            lax.fori_loop(0, _B // 128, vbody, jnp.int32(0))

        pltpu.sync_copy(vbuf, vals_hbm.at[pl.ds(base + bi * _RB, _RB)])

    _start(0, rowA, semA)

    def batch2_body(b2, _):
        bi0 = b2 * 2

        _wait(bi0, rowA, semA)
        _start(bi0 + 1, rowB, semB)
        _do_batch(bi0, rowA)

        _wait(bi0 + 1, rowB, semB)

        @pl.when(bi0 + 2 < nb)
        def _():
            _start(bi0 + 2, rowA, semA)

        _do_batch(bi0 + 1, rowB)
        return 0

    lax.fori_loop(0, nb // 2, batch2_body, 0)


@functools.cache
def _make_sc_compact():
    return pl.kernel(
        _sc_compact_body,
        out_type=[
            jax.ShapeDtypeStruct((_B, _CAP), jnp.float32),
        ],
        mesh=plsc.VectorSubcoreMesh(core_axis_name="c", subcore_axis_name="s"),
        compiler_params=pltpu.CompilerParams(needs_layout_passes=False),
        scratch_types=[
            pltpu.VMEM((_RB, _B), jnp.float32),
            pltpu.VMEM((_RB, _B), jnp.float32),
            pltpu.VMEM((_RB, _CAP), jnp.float32),
            pltpu.SemaphoreType.DMA,
            pltpu.SemaphoreType.DMA,
        ],
    )


_PR = 256


def _cand_body(x_ref, vals_ref, out_ref, bad_ref, s_ref, acc_ref):
    i = pl.program_id(0)
    x0 = x_ref[...]
    vals = vals_ref[...]
    neginf = jnp.float32(-jnp.inf)

    cols = jax.lax.broadcasted_iota(jnp.int32, (_PR, _B), 1)
    rowg = i * _PR + jax.lax.broadcasted_iota(jnp.int32, (_PR, _B), 0)
    pos = jnp.max(jnp.where(cols == rowg, x0, neginf), axis=1, keepdims=True)
    rank_self = jnp.sum(
        ((x0 > pos) | ((x0 == pos) & (cols < rowg))).astype(jnp.float32),
        axis=1, keepdims=True)
    n0 = jnp.sum((x0 > _TAU0).astype(jnp.float32), axis=1, keepdims=True)

    cpos = jax.lax.broadcasted_iota(jnp.int32, (_PR, _CAP), 1).astype(jnp.float32)
    valid = cpos < n0
    v = jnp.where(valid, vals, neginf)

    loss_row, mask_rows = _iter_topk_loss(v, pos, rank_self, _PR, s_ref)
    bad_row = (n0 < float(_K1)) | (n0 > float(_CAP - 16))

    @pl.when(i == 0)
    def _():
        acc_ref[0] = 0.0
        acc_ref[1] = 0.0
        acc_ref[2] = 0.0

    acc_ref[0] += jnp.sum(loss_row)
    acc_ref[1] += jnp.sum(mask_rows)
    acc_ref[2] += jnp.sum(bad_row.astype(jnp.float32))
    out_ref[...] = jnp.full((1, 1), acc_ref[0] / jnp.maximum(acc_ref[1], 1.0),
                            jnp.float32)
    bad_ref[...] = jnp.full((1, 1), acc_ref[2], jnp.float32)


def _cand_loss(scores, vals):
    return pl.pallas_call(
        _cand_body,
        grid=(_B // _PR,),
        in_specs=[
            pl.BlockSpec((_PR, _B), lambda i: (i, 0)),
            pl.BlockSpec((_PR, _CAP), lambda i: (i, 0)),
        ],
        out_specs=[
            pl.BlockSpec((1, 1), lambda i: (0, 0)),
            pl.BlockSpec((1, 1), lambda i: (0, 0)),
        ],
        out_shape=[
            jax.ShapeDtypeStruct((1, 1), jnp.float32),
            jax.ShapeDtypeStruct((1, 1), jnp.float32),
        ],
        scratch_shapes=[
            pltpu.VMEM((_PR, _CAP), jnp.float32),
            pltpu.SMEM((3,), jnp.float32),
        ],
    )(scores, vals)


@jax.jit
def kernel(scores):
    (vals,) = _make_sc_compact()(scores)
    fast, badf = _cand_loss(scores, vals)
    fast_s = jnp.reshape(fast, ())
    bad = jnp.reshape(badf, ()) > 0.0
    return lax.cond(bad, _dense_loss, lambda s: fast_s, scores)
